# baseline (speedup 1.0000x reference)
import functools

import jax
import jax.numpy as jnp
import numpy as np
from jax import lax
from jax.experimental import pallas as pl
from jax.experimental.pallas import tpu as pltpu
from jax.experimental.pallas import tpu_sc as plsc

_DIM = 64
_SCALE = np.float32(8.0)
_LANES = 16
_ISUB = 128


@functools.lru_cache(maxsize=None)
def _build(B, D, NW, K):
    b_per_w = B // NW
    n_chunks = b_per_w // K
    n_sub = b_per_w // _ISUB
    subk = K // _ISUB
    n_groups = n_chunks // 2
    assert B % NW == 0 and b_per_w % K == 0 and K % _ISUB == 0
    assert n_chunks % 2 == 0

    mesh = plsc.VectorSubcoreMesh(core_axis_name="c", subcore_axis_name="s")

    @functools.partial(
        pl.kernel,
        mesh=mesh,
        out_type=jax.ShapeDtypeStruct((B, D), jnp.float32),
        compiler_params=pltpu.CompilerParams(use_tc_tiling_on_sc=False),
        scratch_types=[
            pltpu.VMEM((n_sub, _ISUB), jnp.int32),
            pltpu.VMEM((K, D), jnp.float32),
            pltpu.VMEM((K, D), jnp.float32),
            pltpu.VMEM((K, D), jnp.float32),
            pltpu.VMEM((K, D), jnp.float32),
            pltpu.SemaphoreType.DMA,
            pltpu.SemaphoreType.DMA,
            pltpu.SemaphoreType.DMA,
            pltpu.SemaphoreType.DMA,
        ],
    )
    def gather_scale(idx_hbm, table_hbm, out_hbm, idx_v,
                     ibuf0, ibuf1, obuf0, obuf1, g0, g1, o0, o1):
        ibufs = (ibuf0, ibuf1)
        obufs = (obuf0, obuf1)
        gsems = (g0, g1)
        osems = (o0, o1)
        wid = lax.axis_index("s") * 2 + lax.axis_index("c")
        base = wid * b_per_w
        pltpu.sync_copy(idx_hbm.at[wid], idx_v)

        def fire(j, p):
            for b in range(subk):
                pltpu.async_copy(
                    table_hbm.at[idx_v.at[j * subk + b]],
                    ibufs[p].at[pl.ds(b * _ISUB, _ISUB)],
                    gsems[p])

        def drain_gather(j, p):
            for b in range(subk):
                pltpu.make_async_copy(
                    table_hbm.at[idx_v.at[j * subk + b]],
                    ibufs[p].at[pl.ds(b * _ISUB, _ISUB)],
                    gsems[p]).wait()

        def scale(p):
            def row_body(r, c):
                for kk in range(D // _LANES):
                    sl = pl.ds(kk * _LANES, _LANES)
                    obufs[p][r, sl] = ibufs[p][r, sl] * _SCALE
                return c
            lax.fori_loop(0, K, row_body, 0, unroll=4)

        fire(0, 0)
        fire(1, 1)

        def group_body(g, carry):
            for p in range(2):
                j = 2 * g + p
                drain_gather(j, p)

                @pl.when(g > 0)
                def _wait_prev_out():
                    pltpu.make_async_copy(
                        obufs[p],
                        out_hbm.at[pl.ds(base + (j - 2) * K, K)],
                        osems[p]).wait()

                scale(p)

                @pl.when(g < n_groups - 1)
                def _prefetch():
                    fire(j + 2, p)

                pltpu.async_copy(
                    obufs[p],
                    out_hbm.at[pl.ds(base + j * K, K)],
                    osems[p])
            return carry

        lax.fori_loop(0, n_groups, group_body, 0)

        for p in range(2):
            j = n_chunks - 2 + p
            pltpu.make_async_copy(
                obufs[p],
                out_hbm.at[pl.ds(base + j * K, K)],
                osems[p]).wait()

    return gather_scale


def kernel(input_tensor, table):
    Bt, S = input_tensor.shape
    V, D = table.shape
    B = Bt * S
    NW = 32
    K = 256
    fn = _build(B, D, NW, K)
    idx = input_tensor.reshape(NW, (B // NW) // _ISUB, _ISUB).astype(jnp.int32)
    out = fn(idx, table)
    return out.reshape(Bt, S, D)
